# Initial kernel scaffold; baseline (speedup 1.0000x reference)
import jax
import jax.numpy as jnp
from jax.experimental import pallas as pl


def kernel(features, weight, clf_W, clf_b, nodes, labels, neigh1, neigh2, neigh3):
    raise NotImplementedError("write your pallas kernel here")


# speedup vs baseline: 1.7959x; 1.7959x over previous
import functools

import jax
import jax.numpy as jnp
from jax import lax
from jax.experimental import pallas as pl
from jax.experimental.pallas import tpu as pltpu
from jax.experimental.pallas import tpu_sc as plsc

FEAT = 128
EMBED = 64
NB = 4096
KTOT = 96
NC, NS, L = 2, 16, 16
NW = NC * NS
BPW = NB // NW
VPR = FEAT // L
SCALE = 0.5 / 32.0

_mesh = plsc.VectorSubcoreMesh(
    core_axis_name="c", subcore_axis_name="s", num_cores=NC, num_subcores=NS
)


def _sc_body(feat_hbm, nodes_hbm, nidx_hbm, self_out, x_out,
             nidx_v, sidx_v, self_v, rows0_v, rows1_v, x_v,
             sem_self, sem0, sem1):
    wid = lax.axis_index("s") * NC + lax.axis_index("c")
    base = wid * BPW

    pltpu.sync_copy(nodes_hbm.at[pl.ds(base, BPW)], sidx_v)
    pltpu.sync_copy(nidx_hbm.at[pl.ds(base * KTOT, BPW * KTOT)], nidx_v)

    self_cp = pltpu.async_copy(feat_hbm.at[sidx_v], self_v, sem_self)

    pltpu.async_copy(feat_hbm.at[nidx_v.at[pl.ds(0, KTOT)]], rows0_v, sem0)
    pltpu.async_copy(feat_hbm.at[nidx_v.at[pl.ds(KTOT, KTOT)]], rows1_v, sem1)

    self_cp.wait()

    def step(jj, _):
        for b, (rows_v, sem) in enumerate(((rows0_v, sem0), (rows1_v, sem1))):
            n = jj * 2 + b
            pltpu.make_async_copy(
                feat_hbm.at[nidx_v.at[pl.ds(0, KTOT)]], rows_v, sem
            ).wait()
            def red(r, accs):
                out = accs
                for u in range(4):
                    out = tuple(
                        out[k] + rows_v[r * 4 + u, pl.ds(k * L, L)]
                        for k in range(VPR)
                    )
                return out
            accs = lax.fori_loop(
                0, KTOT // 4, red,
                tuple(jnp.zeros((L,), jnp.float32) for _ in range(VPR)),
            )
            @pl.when(n + 2 < BPW)
            def _():
                pltpu.async_copy(
                    feat_hbm.at[nidx_v.at[pl.ds((n + 2) * KTOT, KTOT)]],
                    rows_v, sem,
                )
            for k in range(VPR):
                x_v[n, pl.ds(k * L, L)] = (
                    self_v[n, pl.ds(k * L, L)] + SCALE * accs[k]
                )
        return 0

    lax.fori_loop(0, BPW // 2, step, 0)

    pltpu.sync_copy(self_v, self_out.at[pl.ds(base, BPW)])
    pltpu.sync_copy(x_v, x_out.at[pl.ds(base, BPW)])


_sc_gather = functools.partial(
    pl.kernel,
    out_type=[
        jax.ShapeDtypeStruct((NB, FEAT), jnp.float32),
        jax.ShapeDtypeStruct((NB, FEAT), jnp.float32),
    ],
    mesh=_mesh,
    scratch_types=[
        pltpu.VMEM((BPW * KTOT,), jnp.int32),
        pltpu.VMEM((BPW,), jnp.int32),
        pltpu.VMEM((BPW, FEAT), jnp.float32),
        pltpu.VMEM((KTOT, FEAT), jnp.float32),
        pltpu.VMEM((KTOT, FEAT), jnp.float32),
        pltpu.VMEM((BPW, FEAT), jnp.float32),
        pltpu.SemaphoreType.DMA,
        pltpu.SemaphoreType.DMA,
        pltpu.SemaphoreType.DMA,
    ],
)(_sc_body)


def _tc_body(x_ref, self_ref, w_ref, cw_ref, cb_ref, comb_ref, scores_ref):
    comb = lax.dot_general(
        w_ref[...], x_ref[...], (((1,), (1,)), ((), ())),
        preferred_element_type=jnp.float32,
    )
    comb_ref[...] = jnp.maximum(comb, 0.0)
    scores_ref[...] = (
        lax.dot_general(
            self_ref[...], cw_ref[...], (((1,), (0,)), ((), ())),
            preferred_element_type=jnp.float32,
        )
        + cb_ref[...]
    )


_tc_dense = pl.pallas_call(
    _tc_body,
    out_shape=(
        jax.ShapeDtypeStruct((EMBED, NB), jnp.float32),
        jax.ShapeDtypeStruct((NB, 2), jnp.float32),
    ),
)


def kernel(features, weight, clf_W, clf_b, nodes, labels, neigh1, neigh2, neigh3):
    del labels
    nidx = jnp.reshape(
        jnp.concatenate(
            [neigh1.astype(jnp.int32), neigh2.astype(jnp.int32),
             neigh3.astype(jnp.int32)], axis=1,
        ),
        (-1,),
    )
    self_rows, x = _sc_gather(features, nodes.astype(jnp.int32), nidx)
    combined, scores = _tc_dense(
        x, self_rows, weight, clf_W, jnp.reshape(clf_b, (1, 2))
    )
    return combined, scores


# speedup vs baseline: 2.5156x; 1.4008x over previous
import functools

import jax
import jax.numpy as jnp
from jax import lax
from jax.experimental import pallas as pl
from jax.experimental.pallas import tpu as pltpu
from jax.experimental.pallas import tpu_sc as plsc

FEAT = 128
EMBED = 64
NB = 4096
KTOT = 96
NC, NS, L = 2, 16, 16
NW = NC * NS
BPW = NB // NW
VPR = FEAT // L
SCALE = 0.5 / 32.0

_mesh = plsc.VectorSubcoreMesh(
    core_axis_name="c", subcore_axis_name="s", num_cores=NC, num_subcores=NS
)


def _sc_body(feat_hbm, nodes_hbm, nidx_hbm, self_out, x_out,
             nidx_v, sidx_v, self_v, acc_v, sem_self, sem_acc):
    wid = lax.axis_index("s") * NC + lax.axis_index("c")
    base = wid * BPW

    pltpu.sync_copy(nodes_hbm.at[pl.ds(base, BPW)], sidx_v)
    pltpu.sync_copy(nidx_hbm.at[wid], nidx_v)

    self_cp = pltpu.async_copy(feat_hbm.at[sidx_v], self_v, sem_self)

    zeros = jnp.zeros((L,), jnp.float32)

    def zero_row(n, _):
        for k in range(VPR):
            acc_v[n, pl.ds(k * L, L)] = zeros
        return 0

    lax.fori_loop(0, BPW, zero_row, 0)

    def fire(p, _):
        pltpu.async_copy(feat_hbm.at[nidx_v.at[p]], acc_v, sem_acc, add=True)
        return 0

    lax.fori_loop(0, KTOT, fire, 0)

    def drain(p, _):
        pltpu.make_async_copy(feat_hbm.at[nidx_v.at[0]], acc_v, sem_acc).wait()
        return 0

    lax.fori_loop(0, KTOT, drain, 0)

    self_cp.wait()

    def combine(n, _):
        for k in range(VPR):
            acc_v[n, pl.ds(k * L, L)] = (
                self_v[n, pl.ds(k * L, L)] + SCALE * acc_v[n, pl.ds(k * L, L)]
            )
        return 0

    lax.fori_loop(0, BPW, combine, 0)

    pltpu.sync_copy(self_v, self_out.at[pl.ds(base, BPW)])
    pltpu.sync_copy(acc_v, x_out.at[pl.ds(base, BPW)])


_sc_gather = functools.partial(
    pl.kernel,
    out_type=[
        jax.ShapeDtypeStruct((NB, FEAT), jnp.float32),
        jax.ShapeDtypeStruct((NB, FEAT), jnp.float32),
    ],
    mesh=_mesh,
    scratch_types=[
        pltpu.VMEM((KTOT, BPW), jnp.int32),
        pltpu.VMEM((BPW,), jnp.int32),
        pltpu.VMEM((BPW, FEAT), jnp.float32),
        pltpu.VMEM((BPW, FEAT), jnp.float32),
        pltpu.SemaphoreType.DMA,
        pltpu.SemaphoreType.DMA,
    ],
)(_sc_body)


def _tc_body(x_ref, self_ref, w_ref, cw_ref, cb_ref, comb_ref, scores_ref):
    comb = lax.dot_general(
        w_ref[...], x_ref[...], (((1,), (1,)), ((), ())),
        preferred_element_type=jnp.float32,
    )
    comb_ref[...] = jnp.maximum(comb, 0.0)
    scores_ref[...] = (
        lax.dot_general(
            self_ref[...], cw_ref[...], (((1,), (0,)), ((), ())),
            preferred_element_type=jnp.float32,
        )
        + cb_ref[...]
    )


_tc_dense = pl.pallas_call(
    _tc_body,
    out_shape=(
        jax.ShapeDtypeStruct((EMBED, NB), jnp.float32),
        jax.ShapeDtypeStruct((NB, 2), jnp.float32),
    ),
)


def kernel(features, weight, clf_W, clf_b, nodes, labels, neigh1, neigh2, neigh3):
    del labels
    rel = [
        jnp.transpose(n.astype(jnp.int32).reshape(NW, BPW, KTOT // 3), (0, 2, 1))
        for n in (neigh1, neigh2, neigh3)
    ]
    nidx = jnp.concatenate(rel, axis=1)
    self_rows, x = _sc_gather(features, nodes.astype(jnp.int32), nidx)
    combined, scores = _tc_dense(
        x, self_rows, weight, clf_W, jnp.reshape(clf_b, (1, 2))
    )
    return combined, scores


# speedup vs baseline: 2.5275x; 1.0047x over previous
import functools

import jax
import jax.numpy as jnp
from jax import lax
from jax.experimental import pallas as pl
from jax.experimental.pallas import tpu as pltpu
from jax.experimental.pallas import tpu_sc as plsc

FEAT = 128
EMBED = 64
NB = 4096
KTOT = 96
NC, NS, L = 2, 16, 16
NW = NC * NS
BPW = NB // NW
VPR = FEAT // L
SCALE = 0.5 / 32.0

_mesh = plsc.VectorSubcoreMesh(
    core_axis_name="c", subcore_axis_name="s", num_cores=NC, num_subcores=NS
)


def _sc_body(feat_hbm, nodes_hbm, nidx_hbm, self_out, x_out,
             nidx_v, sidx_v, self_v, acc_v, sem_self, sem_acc):
    wid = lax.axis_index("s") * NC + lax.axis_index("c")
    base = wid * BPW

    pltpu.sync_copy(nodes_hbm.at[pl.ds(base, BPW)], sidx_v)
    pltpu.sync_copy(nidx_hbm.at[wid], nidx_v)

    self_cp = pltpu.async_copy(feat_hbm.at[sidx_v], self_v, sem_self)

    zeros = jnp.zeros((L,), jnp.float32)

    def zero_row(n, _):
        for k in range(VPR):
            acc_v[n, pl.ds(k * L, L)] = zeros
        return 0

    lax.fori_loop(0, BPW, zero_row, 0)

    def fire(p, _):
        pltpu.async_copy(feat_hbm.at[nidx_v.at[p]], acc_v, sem_acc, add=True)
        return 0

    lax.fori_loop(0, KTOT, fire, 0)

    def drain(p, _):
        pltpu.make_async_copy(feat_hbm.at[nidx_v.at[0]], acc_v, sem_acc).wait()
        return 0

    lax.fori_loop(0, KTOT, drain, 0)

    self_cp.wait()

    def combine(n, _):
        for k in range(VPR):
            acc_v[n, pl.ds(k * L, L)] = (
                self_v[n, pl.ds(k * L, L)] + SCALE * acc_v[n, pl.ds(k * L, L)]
            )
        return 0

    lax.fori_loop(0, BPW, combine, 0)

    pltpu.sync_copy(self_v, self_out.at[pl.ds(base, BPW)])
    pltpu.sync_copy(acc_v, x_out.at[pl.ds(base, BPW)])


_sc_gather = functools.partial(
    pl.kernel,
    out_type=[
        jax.ShapeDtypeStruct((NB, FEAT), jnp.float32),
        jax.ShapeDtypeStruct((NB, FEAT), jnp.float32),
    ],
    mesh=_mesh,
    scratch_types=[
        pltpu.VMEM((KTOT, BPW), jnp.int32),
        pltpu.VMEM((BPW,), jnp.int32),
        pltpu.VMEM((BPW, FEAT), jnp.float32),
        pltpu.VMEM((BPW, FEAT), jnp.float32),
        pltpu.SemaphoreType.DMA,
        pltpu.SemaphoreType.DMA,
    ],
)(_sc_body)


def _tc_body(x_ref, self_ref, w_ref, cw_ref, cb_ref, comb_ref, scores_ref):
    comb = lax.dot_general(
        w_ref[...], x_ref[...], (((1,), (1,)), ((), ())),
        preferred_element_type=jnp.float32,
    )
    comb_ref[...] = jnp.maximum(comb, 0.0)
    scores_ref[...] = (
        lax.dot_general(
            self_ref[...], cw_ref[...], (((1,), (0,)), ((), ())),
            preferred_element_type=jnp.float32,
        )
        + cb_ref[...]
    )


_tc_dense = pl.pallas_call(
    _tc_body,
    out_shape=(
        jax.ShapeDtypeStruct((EMBED, NB), jnp.float32),
        jax.ShapeDtypeStruct((NB, 2), jnp.float32),
    ),
)


def kernel(features, weight, clf_W, clf_b, nodes, labels, neigh1, neigh2, neigh3):
    del labels
    nidx = (
        jnp.stack(
            [neigh1.astype(jnp.int32), neigh2.astype(jnp.int32),
             neigh3.astype(jnp.int32)], axis=0,
        )
        .reshape(3, NW, BPW, KTOT // 3)
        .transpose(1, 0, 3, 2)
        .reshape(NW, KTOT, BPW)
    )
    self_rows, x = _sc_gather(features, nodes.astype(jnp.int32), nidx)
    combined, scores = _tc_dense(
        x, self_rows, weight, clf_W, jnp.reshape(clf_b, (1, 2))
    )
    return combined, scores


# speedup vs baseline: 2.5304x; 1.0012x over previous
import functools

import jax
import jax.numpy as jnp
from jax import lax
from jax.experimental import pallas as pl
from jax.experimental.pallas import tpu as pltpu
from jax.experimental.pallas import tpu_sc as plsc

FEAT = 128
EMBED = 64
NB = 4096
KTOT = 96
NC, NS, L = 2, 16, 16
NW = NC * NS
BPW = NB // NW
VPR = FEAT // L
SCALE = 0.5 / 32.0

_mesh = plsc.VectorSubcoreMesh(
    core_axis_name="c", subcore_axis_name="s", num_cores=NC, num_subcores=NS
)


def _sc_body(feat_hbm, nodes_hbm, nidx_hbm, self_out, x_out,
             nidx_v, sidx_v, self_v, acc_v, sem_self, sem_acc, sem_idx):
    wid = lax.axis_index("s") * NC + lax.axis_index("c")
    base = wid * BPW

    pltpu.sync_copy(nodes_hbm.at[pl.ds(base, BPW)], sidx_v)
    idx_cp = pltpu.async_copy(nidx_hbm.at[wid], nidx_v, sem_idx)

    self_cp = pltpu.async_copy(feat_hbm.at[sidx_v], self_v, sem_self)

    zeros = jnp.zeros((L,), jnp.float32)

    def zero_row(n, _):
        for k in range(VPR):
            acc_v[n, pl.ds(k * L, L)] = zeros
        return 0

    lax.fori_loop(0, BPW, zero_row, 0)
    idx_cp.wait()

    def fire(p, _):
        pltpu.async_copy(feat_hbm.at[nidx_v.at[p]], acc_v, sem_acc, add=True)
        return 0

    lax.fori_loop(0, KTOT, fire, 0)

    def drain(p, _):
        pltpu.make_async_copy(feat_hbm.at[nidx_v.at[0]], acc_v, sem_acc).wait()
        return 0

    lax.fori_loop(0, KTOT, drain, 0)

    self_cp.wait()

    pltpu.sync_copy(self_v, self_out.at[pl.ds(base, BPW)])
    pltpu.sync_copy(acc_v, x_out.at[pl.ds(base, BPW)])


_sc_gather = functools.partial(
    pl.kernel,
    out_type=[
        jax.ShapeDtypeStruct((NB, FEAT), jnp.float32),
        jax.ShapeDtypeStruct((NB, FEAT), jnp.float32),
    ],
    mesh=_mesh,
    scratch_types=[
        pltpu.VMEM((KTOT, BPW), jnp.int32),
        pltpu.VMEM((BPW,), jnp.int32),
        pltpu.VMEM((BPW, FEAT), jnp.float32),
        pltpu.VMEM((BPW, FEAT), jnp.float32),
        pltpu.SemaphoreType.DMA,
        pltpu.SemaphoreType.DMA,
        pltpu.SemaphoreType.DMA,
    ],
)(_sc_body)


def _tc_body(acc_ref, self_ref, w_ref, cw_ref, cb_ref, comb_ref, scores_ref):
    x = self_ref[...] + SCALE * acc_ref[...]
    comb = lax.dot_general(
        w_ref[...], x, (((1,), (1,)), ((), ())),
        preferred_element_type=jnp.float32,
    )
    comb_ref[...] = jnp.maximum(comb, 0.0)
    scores_ref[...] = (
        lax.dot_general(
            self_ref[...], cw_ref[...], (((1,), (0,)), ((), ())),
            preferred_element_type=jnp.float32,
        )
        + cb_ref[...]
    )


_tc_dense = pl.pallas_call(
    _tc_body,
    out_shape=(
        jax.ShapeDtypeStruct((EMBED, NB), jnp.float32),
        jax.ShapeDtypeStruct((NB, 2), jnp.float32),
    ),
)


def kernel(features, weight, clf_W, clf_b, nodes, labels, neigh1, neigh2, neigh3):
    del labels
    nidx = (
        jnp.stack(
            [neigh1.astype(jnp.int32), neigh2.astype(jnp.int32),
             neigh3.astype(jnp.int32)], axis=0,
        )
        .reshape(3, NW, BPW, KTOT // 3)
        .transpose(1, 0, 3, 2)
        .reshape(NW, KTOT, BPW)
    )
    self_rows, x = _sc_gather(features, nodes.astype(jnp.int32), nidx)
    combined, scores = _tc_dense(
        x, self_rows, weight, clf_W, jnp.reshape(clf_b, (1, 2))
    )
    return combined, scores


# speedup vs baseline: 2.5320x; 1.0006x over previous
import functools

import jax
import jax.numpy as jnp
from jax import lax
from jax.experimental import pallas as pl
from jax.experimental.pallas import tpu as pltpu
from jax.experimental.pallas import tpu_sc as plsc

FEAT = 128
EMBED = 64
NB = 4096
KTOT = 96
NC, NS, L = 2, 16, 16
NW = NC * NS
BPW = NB // NW
VPR = FEAT // L
SCALE = 0.5 / 32.0

_mesh = plsc.VectorSubcoreMesh(
    core_axis_name="c", subcore_axis_name="s", num_cores=NC, num_subcores=NS
)


def _sc_body(feat_hbm, nodes_hbm, nidx_hbm, self_out, x_out,
             nidx_v, sidx_v, self_v, acc_v, sem_self, sem_acc, sem_idx):
    wid = lax.axis_index("s") * NC + lax.axis_index("c")
    base = wid * BPW

    pltpu.sync_copy(nodes_hbm.at[pl.ds(base, BPW)], sidx_v)
    idx_cp = pltpu.async_copy(nidx_hbm.at[wid], nidx_v, sem_idx)

    self_cp = pltpu.async_copy(feat_hbm.at[sidx_v], self_v, sem_self)

    zeros = jnp.zeros((L,), jnp.float32)

    def zero_row(n, _):
        for k in range(VPR):
            acc_v[n, pl.ds(k * L, L)] = zeros
        return 0

    lax.fori_loop(0, BPW, zero_row, 0)
    idx_cp.wait()

    def fire(p, _):
        pltpu.async_copy(feat_hbm.at[nidx_v.at[p]], acc_v, sem_acc, add=True)
        return 0

    lax.fori_loop(0, KTOT, fire, 0)

    self_cp.wait()
    pltpu.sync_copy(self_v, self_out.at[pl.ds(base, BPW)])

    def drain(p, _):
        pltpu.make_async_copy(feat_hbm.at[nidx_v.at[0]], acc_v, sem_acc).wait()
        return 0

    lax.fori_loop(0, KTOT, drain, 0)

    pltpu.sync_copy(acc_v, x_out.at[pl.ds(base, BPW)])


_sc_gather = functools.partial(
    pl.kernel,
    out_type=[
        jax.ShapeDtypeStruct((NB, FEAT), jnp.float32),
        jax.ShapeDtypeStruct((NB, FEAT), jnp.float32),
    ],
    mesh=_mesh,
    scratch_types=[
        pltpu.VMEM((KTOT, BPW), jnp.int32),
        pltpu.VMEM((BPW,), jnp.int32),
        pltpu.VMEM((BPW, FEAT), jnp.float32),
        pltpu.VMEM((BPW, FEAT), jnp.float32),
        pltpu.SemaphoreType.DMA,
        pltpu.SemaphoreType.DMA,
        pltpu.SemaphoreType.DMA,
    ],
)(_sc_body)


def _tc_body(acc_ref, self_ref, w_ref, cw_ref, cb_ref, comb_ref, scores_ref):
    x = self_ref[...] + SCALE * acc_ref[...]
    comb = lax.dot_general(
        w_ref[...], x, (((1,), (1,)), ((), ())),
        preferred_element_type=jnp.float32,
    )
    comb_ref[...] = jnp.maximum(comb, 0.0)
    scores_ref[...] = (
        lax.dot_general(
            self_ref[...], cw_ref[...], (((1,), (0,)), ((), ())),
            preferred_element_type=jnp.float32,
        )
        + cb_ref[...]
    )


_tc_dense = pl.pallas_call(
    _tc_body,
    out_shape=(
        jax.ShapeDtypeStruct((EMBED, NB), jnp.float32),
        jax.ShapeDtypeStruct((NB, 2), jnp.float32),
    ),
)


def kernel(features, weight, clf_W, clf_b, nodes, labels, neigh1, neigh2, neigh3):
    del labels
    nidx = (
        jnp.stack(
            [neigh1.astype(jnp.int32), neigh2.astype(jnp.int32),
             neigh3.astype(jnp.int32)], axis=0,
        )
        .reshape(3, NW, BPW, KTOT // 3)
        .transpose(1, 0, 3, 2)
        .reshape(NW, KTOT, BPW)
    )
    self_rows, x = _sc_gather(features, nodes.astype(jnp.int32), nidx)
    combined, scores = _tc_dense(
        x, self_rows, weight, clf_W, jnp.reshape(clf_b, (1, 2))
    )
    return combined, scores
